# baseline (speedup 1.0000x reference)
import jax
import jax.numpy as jnp
from jax.experimental import pallas as pl


def _probe3(x_ref, out_ref):
    @pl.when((pl.program_id(0) == 0) & (pl.program_id(1) == 0))
    def _():
        out_ref[...] = jnp.zeros((1, 1), jnp.float32)
    out_ref[...] += jnp.sum(x_ref[0]).reshape(1, 1)


@jax.jit
def kernel(confidence, predicted_locations, gt_labels, gt_locations):
    B, P, C = confidence.shape
    conf_pad = jnp.pad(confidence, ((0, 0), (0, 0), (0, 128 - C)))
    n = 4
    Pc = 2184
    s = pl.pallas_call(
        _probe3,
        grid=(B, n),
        in_specs=[pl.BlockSpec((1, Pc, 128), lambda b, i: (b, i, 0))],
        out_specs=pl.BlockSpec((1, 1), lambda b, i: (0, 0)),
        out_shape=jax.ShapeDtypeStruct((1, 1), jnp.float32),
    )(conf_pad)
    t = s[0, 0]
    return (t, t)


# speedup vs baseline: 3.6018x; 3.6018x over previous
import jax
import jax.numpy as jnp
from jax.experimental import pallas as pl
from jax.experimental.pallas import tpu as pltpu

_NEG_POS_RATIO = 3.0


def _mbloss_kernel(conf_ref, lab_ref, ploc_ref, gloc_ref,
                   out0_ref, out1_ref,
                   nv_ref, npos_ref, posce_ref, sl1_ref):
    b = pl.program_id(0)
    B = pl.num_programs(0)

    x = conf_ref[0]
    C, P = x.shape
    s = jnp.sum(jnp.exp(x), axis=0, keepdims=True)
    lse = jnp.log(s)

    lab = lab_ref[pl.ds(b, 1), :]
    pos = lab > 0

    cls = jax.lax.broadcasted_iota(jnp.int32, x.shape, 0)
    clabel = jnp.sum(jnp.where(cls == lab, x, 0.0), axis=0, keepdims=True)
    v = lse - clabel

    nv_ref[pl.ds(b, 1), :] = jnp.where(pos, -1.0, v)

    npos_b = jnp.sum(jnp.where(pos, 1.0, 0.0))
    posce_b = jnp.sum(jnp.where(pos, v, 0.0))
    d = ploc_ref[0] - gloc_ref[0]
    ad = jnp.abs(d)
    sl1 = jnp.where(ad < 1.0, 0.5 * d * d, ad - 0.5)
    sl1_b = jnp.sum(jnp.where(pos, sl1, 0.0))

    npos_ref[pl.ds(b, 1), :] = jnp.full((1, 128), npos_b, jnp.float32)
    posce_ref[pl.ds(b, 1), :] = jnp.full((1, 128), posce_b, jnp.float32)
    sl1_ref[pl.ds(b, 1), :] = jnp.full((1, 128), sl1_b, jnp.float32)

    @pl.when(b == B - 1)
    def _finalize():
        nv = nv_ref[...]
        npos = npos_ref[:, 0:1]
        k = jnp.minimum(_NEG_POS_RATIO * npos, float(P) - npos)
        ki = k.astype(jnp.int32)

        iv = jax.lax.bitcast_convert_type(nv, jnp.int32)
        t = jnp.zeros((nv.shape[0], 1), jnp.int32)
        for bit in range(30, -1, -1):
            t2 = t | (1 << bit)
            cnt = jnp.sum((iv >= t2).astype(jnp.int32), axis=1, keepdims=True)
            t = jnp.where(cnt >= ki, t2, t)
        vk = jax.lax.bitcast_convert_type(t, jnp.float32)
        gt = iv > t
        cnt_gt = jnp.sum(gt.astype(jnp.float32), axis=1, keepdims=True)
        sum_gt = jnp.sum(jnp.where(gt, nv, 0.0), axis=1, keepdims=True)
        topk = jnp.where(ki > 0, sum_gt + (k - cnt_gt) * vk, 0.0)

        npos_tot = jnp.sum(npos)
        out0_ref[...] = (jnp.sum(sl1_ref[:, 0:1]) / npos_tot).reshape(1, 1)
        out1_ref[...] = ((jnp.sum(posce_ref[:, 0:1]) + jnp.sum(topk))
                         / npos_tot).reshape(1, 1)


@jax.jit
def kernel(confidence, predicted_locations, gt_labels, gt_locations):
    B, P, C = confidence.shape
    conf_t = jnp.swapaxes(confidence, 1, 2)
    ploc_t = jnp.swapaxes(predicted_locations, 1, 2)
    gloc_t = jnp.swapaxes(gt_locations, 1, 2)
    labels = gt_labels.astype(jnp.int32)

    out0, out1 = pl.pallas_call(
        _mbloss_kernel,
        grid=(B,),
        in_specs=[
            pl.BlockSpec((1, C, P), lambda b: (b, 0, 0)),
            pl.BlockSpec((B, P), lambda b: (0, 0)),
            pl.BlockSpec((1, 4, P), lambda b: (b, 0, 0)),
            pl.BlockSpec((1, 4, P), lambda b: (b, 0, 0)),
        ],
        out_specs=[
            pl.BlockSpec((1, 1), lambda b: (0, 0)),
            pl.BlockSpec((1, 1), lambda b: (0, 0)),
        ],
        out_shape=[
            jax.ShapeDtypeStruct((1, 1), jnp.float32),
            jax.ShapeDtypeStruct((1, 1), jnp.float32),
        ],
        scratch_shapes=[
            pltpu.VMEM((B, P), jnp.float32),
            pltpu.VMEM((B, 128), jnp.float32),
            pltpu.VMEM((B, 128), jnp.float32),
            pltpu.VMEM((B, 128), jnp.float32),
        ],
    )(conf_t, labels, ploc_t, gloc_t)

    return (out0[0, 0], out1[0, 0])
